# baseline (speedup 1.0000x reference)
import jax
import jax.numpy as jnp
from jax.experimental import pallas as pl
from jax.experimental.pallas import tpu as pltpu

_THRESHOLD = 0.5
_NEG_POS_RATIO = 3
_ALPHA = 1.0
_INT_MIN = -2147483648
_NEG_INF = -1e9
_LANES = 1120
_TP = 560


def _make_kernel(P, C, O):
    f32 = jnp.float32
    L = _LANES
    Pp = 8 * L
    TP = _TP
    n_tiles_per_row = L // TP

    def _mbl_kernel(po_ref, ps_ref, bx_ref, pr_ref, lb_ref,
                    loc_ref, npos_ref, cpos_ref, hard_ref):
        i = pl.program_id(0)

        pcx = pr_ref[0]
        pcy = pr_ref[1]
        pw = pr_ref[2]
        ph = pr_ref[3]
        px1 = pcx - pw / 2.0
        py1 = pcy - ph / 2.0
        px2 = pcx + pw / 2.0
        py2 = pcy + ph / 2.0
        parea = (px2 - px1) * (py2 - py1)

        bx = bx_ref[0]
        lb = lb_ref[0, 0]
        iota_s = jax.lax.broadcasted_iota(jnp.int32, (8, L), 0)
        iota_l = jax.lax.broadcasted_iota(jnp.int32, (8, L), 1)
        gidx = iota_s * L + iota_l

        best_iou = None
        best_obj = jnp.zeros((8, L), jnp.int32)
        obj_prior = []
        for j in range(O):
            bx1 = bx[j, 0]
            by1 = bx[j, 1]
            bx2 = bx[j, 2]
            by2 = bx[j, 3]
            inter = (jnp.clip(jnp.minimum(px2, bx2) - jnp.maximum(px1, bx1),
                              0.0, None)
                     * jnp.clip(jnp.minimum(py2, by2) - jnp.maximum(py1, by1),
                                0.0, None))
            barea = (bx2 - bx1) * (by2 - by1)
            ov = inter / (parea + barea - inter)
            if j == 0:
                best_iou = ov
            else:
                upd = ov > best_iou
                best_obj = jnp.where(upd, j, best_obj)
                best_iou = jnp.where(upd, ov, best_iou)
            mj = jnp.max(ov)
            obj_prior.append(jnp.min(jnp.where(ov == mj, gidx, Pp)))

        for j in range(O):
            msk = gidx == obj_prior[j]
            best_obj = jnp.where(msk, j, best_obj)
            best_iou = jnp.where(msk, 1.0, best_iou)

        gx1 = jnp.zeros((8, L), f32)
        gy1 = jnp.zeros((8, L), f32)
        gx2 = jnp.zeros((8, L), f32)
        gy2 = jnp.zeros((8, L), f32)
        lab = jnp.zeros((8, L), jnp.int32)
        for j in range(O):
            sel = best_obj == j
            gx1 = jnp.where(sel, bx[j, 0], gx1)
            gy1 = jnp.where(sel, bx[j, 1], gy1)
            gx2 = jnp.where(sel, bx[j, 2], gx2)
            gy2 = jnp.where(sel, bx[j, 3], gy2)
            lab = jnp.where(sel, lb[j], lab)
        prior_class = jnp.where(best_iou < _THRESHOLD, 0, lab)
        pos = prior_class != 0
        posf = pos.astype(f32)
        n_pos = jnp.sum(posf)

        bcx = (gx1 + gx2) / 2.0
        bcy = (gy1 + gy2) / 2.0
        bw = gx2 - gx1
        bh = gy2 - gy1
        g_cx = (bcx - pcx) / (pw / 10.0)
        g_cy = (bcy - pcy) / (ph / 10.0)
        g_w = jnp.log(bw / pw) * 5.0
        g_h = jnp.log(bh / ph) * 5.0
        loc_sum = (jnp.sum(jnp.abs(po_ref[0, 0] - g_cx) * posf)
                   + jnp.sum(jnp.abs(po_ref[0, 1] - g_cy) * posf)
                   + jnp.sum(jnp.abs(po_ref[0, 2] - g_w) * posf)
                   + jnp.sum(jnp.abs(po_ref[0, 3] - g_h) * posf))

        ident = (jax.lax.broadcasted_iota(jnp.int32, (TP, TP), 0)
                 == jax.lax.broadcasted_iota(jnp.int32, (TP, TP), 1)
                 ).astype(f32)
        iota_ct = jax.lax.broadcasted_iota(jnp.int32, (C, TP), 0)
        iota_row = jax.lax.broadcasted_iota(jnp.int32, (1, TP), 1)
        iota_col = jax.lax.broadcasted_iota(jnp.int32, (TP, 1), 0)
        conf_pos = jnp.float32(0.0)
        neg_rows = []
        for s in range(8):
            row_parts = []
            for u in range(n_tiles_per_row):
                g0 = s * L + u * TP
                ps_t = ps_ref[0, pl.ds(g0, TP), :]
                if g0 + TP > P:
                    ps_t = jnp.where(iota_col + g0 < P, ps_t, 0.0)
                psT = jax.lax.dot_general(
                    ps_t, ident, (((0,), (0,)), ((), ())),
                    preferred_element_type=f32)
                cls_row = prior_class[s:s + 1, u * TP:(u + 1) * TP]
                mx = jnp.max(psT, axis=0, keepdims=True)
                lse = jnp.log(jnp.sum(jnp.exp(psT - mx), axis=0,
                                      keepdims=True)) + mx
                ts = jnp.sum(jnp.where(iota_ct == cls_row, psT, 0.0),
                             axis=0, keepdims=True)
                conf = lse - ts
                pos_r = cls_row != 0
                conf_pos += jnp.sum(jnp.where(pos_r, conf, 0.0))
                valid = (iota_row + g0) < P
                neg_row = jnp.where(
                    jnp.logical_or(pos_r, jnp.logical_not(valid)),
                    _NEG_INF, conf)
                row_parts.append(neg_row)
            neg_rows.append(jnp.concatenate(row_parts, axis=1))
        neg = jnp.concatenate(neg_rows, axis=0)

        k = jnp.minimum((_NEG_POS_RATIO * n_pos).astype(jnp.int32), P)
        s32 = jax.lax.bitcast_convert_type(neg, jnp.int32)
        key = s32 ^ (jnp.right_shift(s32, 31) & 0x7FFFFFFF)
        nn_i = (key >= 0).astype(jnp.int32)
        c_hi = jnp.sum(nn_i)
        use_hi = k <= c_hi
        k2 = jnp.where(use_hi, k, k - c_hi)
        grp = nn_i == jnp.where(use_hi, 1, 0)
        low = key & 0x7FFFFFFF

        def _bit_step(t, p_acc):
            cand = p_acc | jnp.left_shift(jnp.int32(1), 30 - t)
            cnt = jnp.sum(jnp.where(jnp.logical_and(grp, low >= cand), 1, 0))
            return jnp.where(cnt >= k2, cand, p_acc)

        p_acc = jax.lax.fori_loop(0, 31, _bit_step, jnp.int32(0))
        key_k = jnp.where(use_hi, p_acc, p_acc | _INT_MIN)
        vbits = key_k ^ (jnp.right_shift(key_k, 31) & 0x7FFFFFFF)
        v_k = jax.lax.bitcast_convert_type(vbits, f32)
        gt = key > key_k
        cnt_gt = jnp.sum(jnp.where(gt, 1, 0))
        sum_gt = jnp.sum(jnp.where(gt, neg, 0.0))
        hard = sum_gt + (k - cnt_gt).astype(f32) * v_k

        loc_ref[0, 0, 0] = loc_sum
        npos_ref[0, 0, 0] = n_pos
        cpos_ref[0, 0, 0] = conf_pos
        hard_ref[0, 0, 0] = hard

    return _mbl_kernel


def kernel(predicted_offsets, predicted_scores, boxes, priors_cxcy, labels):
    N, P, C = predicted_scores.shape
    O = labels.shape[1]
    L = _LANES
    Pp = 8 * L
    padn = Pp - P

    pr = priors_cxcy
    if padn:
        pad_rows = jnp.tile(jnp.array([[0.5, 0.5, 1.0, 1.0]], jnp.float32),
                            (padn, 1))
        pr = jnp.concatenate([pr, pad_rows], axis=0)
    pr_t = pr.T.reshape(4, 8, L)

    po = predicted_offsets
    if padn:
        po = jnp.pad(po, ((0, 0), (0, padn), (0, 0)))
    po_t = po.transpose(0, 2, 1).reshape(N, 4, 8, L)

    lb3 = labels.astype(jnp.int32).reshape(N, 1, O)

    outs = pl.pallas_call(
        _make_kernel(P, C, O),
        grid=(N,),
        in_specs=[
            pl.BlockSpec((1, 4, 8, L), lambda i: (i, 0, 0, 0)),
            pl.BlockSpec((1, Pp, C), lambda i: (i, 0, 0)),
            pl.BlockSpec((1, O, 4), lambda i: (i, 0, 0)),
            pl.BlockSpec((4, 8, L), lambda i: (0, 0, 0)),
            pl.BlockSpec((1, 1, O), lambda i: (i, 0, 0)),
        ],
        out_specs=[
            pl.BlockSpec(memory_space=pltpu.SMEM,
                         block_shape=(1, 1, 1), index_map=lambda i: (i, 0, 0)),
        ] * 4,
        out_shape=[jax.ShapeDtypeStruct((N, 1, 1), jnp.float32)] * 4,
        compiler_params=pltpu.CompilerParams(
            dimension_semantics=("parallel",)),
    )(po_t, predicted_scores, boxes, pr_t, lb3)
    loc_sum, n_pos_total, conf_pos, hard = [o.sum() for o in outs]
    loc_loss = loc_sum / (n_pos_total * 4.0)
    conf_loss = (conf_pos + hard) / n_pos_total
    return _ALPHA * loc_loss + conf_loss


# speedup vs baseline: 1.4007x; 1.4007x over previous
import jax
import jax.numpy as jnp
from jax.experimental import pallas as pl
from jax.experimental.pallas import tpu as pltpu

_THRESHOLD = 0.5
_NEG_POS_RATIO = 3
_ALPHA = 1.0
_INT_MIN = -2147483648
_NEG_INF = -1e9
_LANES = 1120
_TP = 560


def _make_kernel(P, C, O):
    f32 = jnp.float32
    L = _LANES
    Pp = 8 * L
    TP = _TP
    n_tiles_per_row = L // TP

    def _mbl_kernel(po_ref, ps_ref, bx_ref, pr_ref, lb_ref,
                    loc_ref, npos_ref, cpos_ref, neg_ref):

        pcx = pr_ref[0]
        pcy = pr_ref[1]
        pw = pr_ref[2]
        ph = pr_ref[3]
        px1 = pcx - pw / 2.0
        py1 = pcy - ph / 2.0
        px2 = pcx + pw / 2.0
        py2 = pcy + ph / 2.0
        parea = (px2 - px1) * (py2 - py1)

        bx = bx_ref[0]
        lb = lb_ref[0, 0]
        iota_s = jax.lax.broadcasted_iota(jnp.int32, (8, L), 0)
        iota_l = jax.lax.broadcasted_iota(jnp.int32, (8, L), 1)
        gidx = iota_s * L + iota_l

        best_iou = None
        best_obj = jnp.zeros((8, L), jnp.int32)
        obj_prior = []
        for j in range(O):
            bx1 = bx[j, 0]
            by1 = bx[j, 1]
            bx2 = bx[j, 2]
            by2 = bx[j, 3]
            inter = (jnp.clip(jnp.minimum(px2, bx2) - jnp.maximum(px1, bx1),
                              0.0, None)
                     * jnp.clip(jnp.minimum(py2, by2) - jnp.maximum(py1, by1),
                                0.0, None))
            barea = (bx2 - bx1) * (by2 - by1)
            ov = inter / (parea + barea - inter)
            if j == 0:
                best_iou = ov
            else:
                upd = ov > best_iou
                best_obj = jnp.where(upd, j, best_obj)
                best_iou = jnp.where(upd, ov, best_iou)
            mj = jnp.max(ov)
            obj_prior.append(jnp.min(jnp.where(ov == mj, gidx, Pp)))

        for j in range(O):
            msk = gidx == obj_prior[j]
            best_obj = jnp.where(msk, j, best_obj)
            best_iou = jnp.where(msk, 1.0, best_iou)

        gx1 = jnp.zeros((8, L), f32)
        gy1 = jnp.zeros((8, L), f32)
        gx2 = jnp.zeros((8, L), f32)
        gy2 = jnp.zeros((8, L), f32)
        lab = jnp.zeros((8, L), jnp.int32)
        for j in range(O):
            sel = best_obj == j
            gx1 = jnp.where(sel, bx[j, 0], gx1)
            gy1 = jnp.where(sel, bx[j, 1], gy1)
            gx2 = jnp.where(sel, bx[j, 2], gx2)
            gy2 = jnp.where(sel, bx[j, 3], gy2)
            lab = jnp.where(sel, lb[j], lab)
        prior_class = jnp.where(best_iou < _THRESHOLD, 0, lab)
        pos = prior_class != 0
        posf = pos.astype(f32)
        n_pos = jnp.sum(posf)

        bcx = (gx1 + gx2) / 2.0
        bcy = (gy1 + gy2) / 2.0
        bw = gx2 - gx1
        bh = gy2 - gy1
        g_cx = (bcx - pcx) / (pw / 10.0)
        g_cy = (bcy - pcy) / (ph / 10.0)
        g_w = jnp.log(bw / pw) * 5.0
        g_h = jnp.log(bh / ph) * 5.0
        loc_sum = (jnp.sum(jnp.abs(po_ref[0, 0] - g_cx) * posf)
                   + jnp.sum(jnp.abs(po_ref[0, 1] - g_cy) * posf)
                   + jnp.sum(jnp.abs(po_ref[0, 2] - g_w) * posf)
                   + jnp.sum(jnp.abs(po_ref[0, 3] - g_h) * posf))

        ident = (jax.lax.broadcasted_iota(jnp.int32, (TP, TP), 0)
                 == jax.lax.broadcasted_iota(jnp.int32, (TP, TP), 1)
                 ).astype(f32)
        iota_ct = jax.lax.broadcasted_iota(jnp.int32, (C, TP), 0)
        iota_row = jax.lax.broadcasted_iota(jnp.int32, (1, TP), 1)
        iota_col = jax.lax.broadcasted_iota(jnp.int32, (TP, 1), 0)
        conf_pos = jnp.float32(0.0)
        neg_rows = []
        for s in range(8):
            row_parts = []
            for u in range(n_tiles_per_row):
                g0 = s * L + u * TP
                ps_t = ps_ref[0, pl.ds(g0, TP), :]
                if g0 + TP > P:
                    ps_t = jnp.where(iota_col + g0 < P, ps_t, 0.0)
                psT = jax.lax.dot_general(
                    ps_t, ident, (((0,), (0,)), ((), ())),
                    preferred_element_type=f32)
                cls_row = prior_class[s:s + 1, u * TP:(u + 1) * TP]
                mx = jnp.max(psT, axis=0, keepdims=True)
                lse = jnp.log(jnp.sum(jnp.exp(psT - mx), axis=0,
                                      keepdims=True)) + mx
                ts = jnp.sum(jnp.where(iota_ct == cls_row, psT, 0.0),
                             axis=0, keepdims=True)
                conf = lse - ts
                pos_r = cls_row != 0
                conf_pos += jnp.sum(jnp.where(pos_r, conf, 0.0))
                valid = (iota_row + g0) < P
                neg_row = jnp.where(
                    jnp.logical_or(pos_r, jnp.logical_not(valid)),
                    _NEG_INF, conf)
                row_parts.append(neg_row)
            neg_rows.append(jnp.concatenate(row_parts, axis=1))
        neg_ref[0] = jnp.concatenate(neg_rows, axis=0)

        loc_ref[0, 0, 0] = loc_sum
        npos_ref[0, 0, 0] = n_pos
        cpos_ref[0, 0, 0] = conf_pos

    return _mbl_kernel


def _make_mine_kernel(P, G):
    f32 = jnp.float32

    def _mine(neg_ref, npos_ref, hard_ref):
        neg = neg_ref[...]
        npos = npos_ref[...]
        k = jnp.minimum((_NEG_POS_RATIO * npos).astype(jnp.int32), P)
        s32 = jax.lax.bitcast_convert_type(neg, jnp.int32)
        key = s32 ^ (jnp.right_shift(s32, 31) & 0x7FFFFFFF)
        nn_i = (key >= 0).astype(jnp.int32)
        c_hi = jnp.sum(nn_i, axis=1, keepdims=True)
        use_hi = k <= c_hi
        k2 = jnp.where(use_hi, k, k - c_hi)
        grp = nn_i == jnp.where(use_hi, 1, 0)
        low = key & 0x7FFFFFFF

        def _bit_step(t, p_acc):
            cand = p_acc | jnp.left_shift(jnp.int32(1), 30 - t)
            cnt = jnp.sum(
                jnp.where(jnp.logical_and(grp, low >= cand), 1, 0),
                axis=1, keepdims=True)
            return jnp.where(cnt >= k2, cand, p_acc)

        p_acc = jax.lax.fori_loop(0, 31, _bit_step,
                                  jnp.zeros((G, 1), jnp.int32))
        key_k = jnp.where(use_hi, p_acc, p_acc | _INT_MIN)
        vbits = key_k ^ (jnp.right_shift(key_k, 31) & 0x7FFFFFFF)
        v_k = jax.lax.bitcast_convert_type(vbits, f32)
        gt = key > key_k
        cnt_gt = jnp.sum(jnp.where(gt, 1, 0), axis=1, keepdims=True)
        sum_gt = jnp.sum(jnp.where(gt, neg, 0.0), axis=1, keepdims=True)
        hard = sum_gt + (k - cnt_gt).astype(f32) * v_k
        hard_ref[0, 0, 0] = jnp.sum(hard)

    return _mine


def kernel(predicted_offsets, predicted_scores, boxes, priors_cxcy, labels):
    N, P, C = predicted_scores.shape
    O = labels.shape[1]
    L = _LANES
    Pp = 8 * L
    padn = Pp - P

    pr = priors_cxcy
    if padn:
        pad_rows = jnp.tile(jnp.array([[0.5, 0.5, 1.0, 1.0]], jnp.float32),
                            (padn, 1))
        pr = jnp.concatenate([pr, pad_rows], axis=0)
    pr_t = pr.T.reshape(4, 8, L)

    po = predicted_offsets
    if padn:
        po = jnp.pad(po, ((0, 0), (0, padn), (0, 0)))
    po_t = po.transpose(0, 2, 1).reshape(N, 4, 8, L)

    lb3 = labels.astype(jnp.int32).reshape(N, 1, O)

    loc_o, npos_o, cpos_o, neg_o = pl.pallas_call(
        _make_kernel(P, C, O),
        grid=(N,),
        in_specs=[
            pl.BlockSpec((1, 4, 8, L), lambda i: (i, 0, 0, 0)),
            pl.BlockSpec((1, Pp, C), lambda i: (i, 0, 0)),
            pl.BlockSpec((1, O, 4), lambda i: (i, 0, 0)),
            pl.BlockSpec((4, 8, L), lambda i: (0, 0, 0)),
            pl.BlockSpec((1, 1, O), lambda i: (i, 0, 0)),
        ],
        out_specs=[
            pl.BlockSpec(memory_space=pltpu.SMEM,
                         block_shape=(1, 1, 1), index_map=lambda i: (i, 0, 0)),
        ] * 3 + [
            pl.BlockSpec((1, 8, L), lambda i: (i, 0, 0)),
        ],
        out_shape=[jax.ShapeDtypeStruct((N, 1, 1), jnp.float32)] * 3 + [
            jax.ShapeDtypeStruct((N, 8, L), jnp.float32)],
        compiler_params=pltpu.CompilerParams(
            dimension_semantics=("parallel",)),
    )(po_t, predicted_scores, boxes, pr_t, lb3)

    G = 8 if N % 8 == 0 else N
    NG = N // G
    hard_o = pl.pallas_call(
        _make_mine_kernel(P, G),
        grid=(NG,),
        in_specs=[
            pl.BlockSpec((G, Pp), lambda g: (g, 0)),
            pl.BlockSpec((G, 1), lambda g: (g, 0)),
        ],
        out_specs=pl.BlockSpec(memory_space=pltpu.SMEM,
                               block_shape=(1, 1, 1),
                               index_map=lambda g: (g, 0, 0)),
        out_shape=jax.ShapeDtypeStruct((NG, 1, 1), jnp.float32),
        compiler_params=pltpu.CompilerParams(
            dimension_semantics=("parallel",)),
    )(neg_o.reshape(N, Pp), npos_o.reshape(N, 1))

    loc_sum = loc_o.sum()
    n_pos_total = npos_o.sum()
    conf_pos = cpos_o.sum()
    hard = hard_o.sum()
    loc_loss = loc_sum / (n_pos_total * 4.0)
    conf_loss = (conf_pos + hard) / n_pos_total
    return _ALPHA * loc_loss + conf_loss


# speedup vs baseline: 1.4762x; 1.0539x over previous
import jax
import jax.numpy as jnp
from jax.experimental import pallas as pl
from jax.experimental.pallas import tpu as pltpu

_THRESHOLD = 0.5
_NEG_POS_RATIO = 3
_ALPHA = 1.0
_INT_MIN = -2147483648
_NEG_INF = -1e9
_LANES = 1120
_TP = 560


def _make_kernel(P, C, O):
    f32 = jnp.float32
    L = _LANES
    Pp = 8 * L
    TP = _TP
    n_tiles_per_row = L // TP

    def _mbl_kernel(po_ref, ps_ref, bx_ref, pr_ref, lb_ref,
                    loc_ref, npos_ref, cpos_ref, neg_ref):

        pcx = pr_ref[0]
        pcy = pr_ref[1]
        pw = pr_ref[2]
        ph = pr_ref[3]
        px1 = pcx - pw / 2.0
        py1 = pcy - ph / 2.0
        px2 = pcx + pw / 2.0
        py2 = pcy + ph / 2.0
        parea = (px2 - px1) * (py2 - py1)

        bx = bx_ref[0]
        lb = lb_ref[0, 0]
        iota_s = jax.lax.broadcasted_iota(jnp.int32, (8, L), 0)
        iota_l = jax.lax.broadcasted_iota(jnp.int32, (8, L), 1)
        gidx = iota_s * L + iota_l

        best_iou = None
        best_obj = jnp.zeros((8, L), jnp.int32)
        ovs = []
        for j in range(O):
            bx1 = bx[j, 0]
            by1 = bx[j, 1]
            bx2 = bx[j, 2]
            by2 = bx[j, 3]
            inter = (jnp.clip(jnp.minimum(px2, bx2) - jnp.maximum(px1, bx1),
                              0.0, None)
                     * jnp.clip(jnp.minimum(py2, by2) - jnp.maximum(py1, by1),
                                0.0, None))
            barea = (bx2 - bx1) * (by2 - by1)
            ov = inter / (parea + barea - inter)
            ovs.append(ov)
            if j == 0:
                best_iou = ov
            else:
                upd = ov > best_iou
                best_obj = jnp.where(upd, j, best_obj)
                best_iou = jnp.where(upd, ov, best_iou)
        maxes = [jnp.max(ov) for ov in ovs]
        obj_prior = [jnp.min(jnp.where(ovs[j] == maxes[j], gidx, Pp))
                     for j in range(O)]

        for j in range(O):
            msk = gidx == obj_prior[j]
            best_obj = jnp.where(msk, j, best_obj)
            best_iou = jnp.where(msk, 1.0, best_iou)

        gx1 = jnp.zeros((8, L), f32)
        gy1 = jnp.zeros((8, L), f32)
        gx2 = jnp.zeros((8, L), f32)
        gy2 = jnp.zeros((8, L), f32)
        lab = jnp.zeros((8, L), jnp.int32)
        for j in range(O):
            sel = best_obj == j
            gx1 = jnp.where(sel, bx[j, 0], gx1)
            gy1 = jnp.where(sel, bx[j, 1], gy1)
            gx2 = jnp.where(sel, bx[j, 2], gx2)
            gy2 = jnp.where(sel, bx[j, 3], gy2)
            lab = jnp.where(sel, lb[j], lab)
        prior_class = jnp.where(best_iou < _THRESHOLD, 0, lab)
        pos = prior_class != 0
        posf = pos.astype(f32)
        n_pos = jnp.sum(posf)

        bcx = (gx1 + gx2) / 2.0
        bcy = (gy1 + gy2) / 2.0
        bw = gx2 - gx1
        bh = gy2 - gy1
        g_cx = (bcx - pcx) / (pw / 10.0)
        g_cy = (bcy - pcy) / (ph / 10.0)
        g_w = jnp.log(bw / pw) * 5.0
        g_h = jnp.log(bh / ph) * 5.0
        loc_sum = (jnp.sum(jnp.abs(po_ref[0, 0] - g_cx) * posf)
                   + jnp.sum(jnp.abs(po_ref[0, 1] - g_cy) * posf)
                   + jnp.sum(jnp.abs(po_ref[0, 2] - g_w) * posf)
                   + jnp.sum(jnp.abs(po_ref[0, 3] - g_h) * posf))

        ident = (jax.lax.broadcasted_iota(jnp.int32, (TP, TP), 0)
                 == jax.lax.broadcasted_iota(jnp.int32, (TP, TP), 1)
                 ).astype(f32)
        iota_ct = jax.lax.broadcasted_iota(jnp.int32, (C, TP), 0)
        iota_row = jax.lax.broadcasted_iota(jnp.int32, (1, TP), 1)
        iota_col = jax.lax.broadcasted_iota(jnp.int32, (TP, 1), 0)
        conf_pos = jnp.float32(0.0)
        neg_rows = []
        for s in range(8):
            row_parts = []
            for u in range(n_tiles_per_row):
                g0 = s * L + u * TP
                ps_t = ps_ref[0, pl.ds(g0, TP), :]
                if g0 + TP > P:
                    ps_t = jnp.where(iota_col + g0 < P, ps_t, 0.0)
                psT = jax.lax.dot_general(
                    ps_t, ident, (((0,), (0,)), ((), ())),
                    preferred_element_type=f32)
                cls_row = prior_class[s:s + 1, u * TP:(u + 1) * TP]
                mx = jnp.max(psT, axis=0, keepdims=True)
                lse = jnp.log(jnp.sum(jnp.exp(psT - mx), axis=0,
                                      keepdims=True)) + mx
                ts = jnp.sum(jnp.where(iota_ct == cls_row, psT, 0.0),
                             axis=0, keepdims=True)
                conf = lse - ts
                pos_r = cls_row != 0
                conf_pos += jnp.sum(jnp.where(pos_r, conf, 0.0))
                valid = (iota_row + g0) < P
                neg_row = jnp.where(
                    jnp.logical_or(pos_r, jnp.logical_not(valid)),
                    _NEG_INF, conf)
                row_parts.append(neg_row)
            neg_rows.append(jnp.concatenate(row_parts, axis=1))
        neg_ref[0] = jnp.concatenate(neg_rows, axis=0)

        loc_ref[0, 0, 0] = loc_sum
        npos_ref[0, 0, 0] = n_pos
        cpos_ref[0, 0, 0] = conf_pos

    return _mbl_kernel


def _make_mine_kernel(P, G):
    f32 = jnp.float32

    def _mine(neg_ref, npos_ref, hard_ref):
        neg = neg_ref[...]
        npos = npos_ref[...]
        k = jnp.minimum((_NEG_POS_RATIO * npos).astype(jnp.int32), P)
        s32 = jax.lax.bitcast_convert_type(neg, jnp.int32)
        key = s32 ^ (jnp.right_shift(s32, 31) & 0x7FFFFFFF)
        nn_i = (key >= 0).astype(jnp.int32)
        c_hi = jnp.sum(nn_i, axis=1, keepdims=True)
        use_hi = k <= c_hi
        k2 = jnp.where(use_hi, k, k - c_hi)
        grp = nn_i == jnp.where(use_hi, 1, 0)
        low = key & 0x7FFFFFFF

        def _bit_step(t, p_acc):
            cand = p_acc | jnp.left_shift(jnp.int32(1), 30 - t)
            cnt = jnp.sum(
                jnp.where(jnp.logical_and(grp, low >= cand), 1, 0),
                axis=1, keepdims=True)
            return jnp.where(cnt >= k2, cand, p_acc)

        p_acc = jax.lax.fori_loop(0, 31, _bit_step,
                                  jnp.zeros((G, 1), jnp.int32))
        key_k = jnp.where(use_hi, p_acc, p_acc | _INT_MIN)
        vbits = key_k ^ (jnp.right_shift(key_k, 31) & 0x7FFFFFFF)
        v_k = jax.lax.bitcast_convert_type(vbits, f32)
        gt = key > key_k
        cnt_gt = jnp.sum(jnp.where(gt, 1, 0), axis=1, keepdims=True)
        sum_gt = jnp.sum(jnp.where(gt, neg, 0.0), axis=1, keepdims=True)
        hard = sum_gt + (k - cnt_gt).astype(f32) * v_k
        hard_ref[0, 0, 0] = jnp.sum(hard)

    return _mine


def kernel(predicted_offsets, predicted_scores, boxes, priors_cxcy, labels):
    N, P, C = predicted_scores.shape
    O = labels.shape[1]
    L = _LANES
    Pp = 8 * L
    padn = Pp - P

    pr = priors_cxcy
    if padn:
        pad_rows = jnp.tile(jnp.array([[0.5, 0.5, 1.0, 1.0]], jnp.float32),
                            (padn, 1))
        pr = jnp.concatenate([pr, pad_rows], axis=0)
    pr_t = pr.T.reshape(4, 8, L)

    po = predicted_offsets
    if padn:
        po = jnp.pad(po, ((0, 0), (0, padn), (0, 0)))
    po_t = po.transpose(0, 2, 1).reshape(N, 4, 8, L)

    lb3 = labels.astype(jnp.int32).reshape(N, 1, O)

    loc_o, npos_o, cpos_o, neg_o = pl.pallas_call(
        _make_kernel(P, C, O),
        grid=(N,),
        in_specs=[
            pl.BlockSpec((1, 4, 8, L), lambda i: (i, 0, 0, 0)),
            pl.BlockSpec((1, Pp, C), lambda i: (i, 0, 0)),
            pl.BlockSpec((1, O, 4), lambda i: (i, 0, 0)),
            pl.BlockSpec((4, 8, L), lambda i: (0, 0, 0)),
            pl.BlockSpec((1, 1, O), lambda i: (i, 0, 0)),
        ],
        out_specs=[
            pl.BlockSpec(memory_space=pltpu.SMEM,
                         block_shape=(1, 1, 1), index_map=lambda i: (i, 0, 0)),
        ] * 3 + [
            pl.BlockSpec((1, 8, L), lambda i: (i, 0, 0)),
        ],
        out_shape=[jax.ShapeDtypeStruct((N, 1, 1), jnp.float32)] * 3 + [
            jax.ShapeDtypeStruct((N, 8, L), jnp.float32)],
        compiler_params=pltpu.CompilerParams(
            dimension_semantics=("parallel",)),
    )(po_t, predicted_scores, boxes, pr_t, lb3)

    G = 16 if N % 16 == 0 else (8 if N % 8 == 0 else N)
    NG = N // G
    hard_o = pl.pallas_call(
        _make_mine_kernel(P, G),
        grid=(NG,),
        in_specs=[
            pl.BlockSpec((G, Pp), lambda g: (g, 0)),
            pl.BlockSpec((G, 1), lambda g: (g, 0)),
        ],
        out_specs=pl.BlockSpec(memory_space=pltpu.SMEM,
                               block_shape=(1, 1, 1),
                               index_map=lambda g: (g, 0, 0)),
        out_shape=jax.ShapeDtypeStruct((NG, 1, 1), jnp.float32),
        compiler_params=pltpu.CompilerParams(
            dimension_semantics=("parallel",)),
    )(neg_o.reshape(N, Pp), npos_o.reshape(N, 1))

    loc_sum = loc_o.sum()
    n_pos_total = npos_o.sum()
    conf_pos = cpos_o.sum()
    hard = hard_o.sum()
    loc_loss = loc_sum / (n_pos_total * 4.0)
    conf_loss = (conf_pos + hard) / n_pos_total
    return _ALPHA * loc_loss + conf_loss
